# baseline (speedup 1.0000x reference)
import functools

import jax
import jax.numpy as jnp
from jax import lax
from jax.experimental import pallas as pl
from jax.experimental.pallas import tpu as pltpu
from jax.experimental.pallas import tpu_sc as plsc

B = 16384
D = 32
L = 16
NC, NS = 2, 16
NW = NC * NS
BPW = B // NW
CHUNK = 128
NCHUNK = BPW // CHUNK
CGROUPS = CHUNK // L
VP = 250000

_mesh = plsc.VectorSubcoreMesh(core_axis_name="c", subcore_axis_name="s")


@functools.partial(
    pl.kernel,
    mesh=_mesh,
    compiler_params=pltpu.CompilerParams(needs_layout_passes=False),
    out_type=(
        jax.ShapeDtypeStruct((B,), jnp.float32),
        jax.ShapeDtypeStruct((B,), jnp.float32),
    ),
    scratch_types=[
        pltpu.VMEM((BPW,), jnp.int32),
        pltpu.VMEM((BPW,), jnp.int32),
        pltpu.VMEM((BPW,), jnp.int32),
        pltpu.VMEM((CHUNK,), jnp.int32),
        pltpu.VMEM((CHUNK,), jnp.int32),
        pltpu.VMEM((CHUNK,), jnp.int32),
        pltpu.VMEM((CHUNK, 128), jnp.float32),
        pltpu.VMEM((CHUNK, 128), jnp.float32),
        pltpu.VMEM((CHUNK, 128), jnp.float32),
        pltpu.VMEM((BPW,), jnp.float32),
        pltpu.VMEM((BPW,), jnp.float32),
        pltpu.SemaphoreType.DMA,
    ],
)
def _mf_kernel(users_hbm, pos_hbm, neg_hbm, utab_hbm, itab_hbm,
               pout_hbm, nout_hbm,
               uidx, pidx, nidx, upk, ppk, npk, ubuf, pbuf, nbuf,
               pout, nout, sem):
    wid = lax.axis_index("s") * NC + lax.axis_index("c")
    base = wid * BPW

    pltpu.sync_copy(users_hbm.at[wid], uidx)
    pltpu.sync_copy(pos_hbm.at[wid], pidx)
    pltpu.sync_copy(neg_hbm.at[wid], nidx)

    tabs = ((uidx, upk, utab_hbm, ubuf),
            (pidx, ppk, itab_hbm, pbuf),
            (nidx, npk, itab_hbm, nbuf))

    def chunk_body(c, carry):
        c0 = c * CHUNK
        for idx_ref, pk, _, _ in tabs:
            for k in range(CGROUPS):
                v = idx_ref[pl.ds(c0 + k * L, L)]
                pk[pl.ds(k * L, L)] = v >> 2
        copies = [pltpu.async_copy(tab.at[pk], buf, sem)
                  for _, pk, tab, buf in tabs]
        for cp in copies:
            cp.wait()

        for k in range(CGROUPS):
            row0 = c0 + k * L
            ridx = k * L + lax.iota(jnp.int32, L)
            ucol = (uidx[pl.ds(row0, L)] & 3) * D
            pcol = (pidx[pl.ds(row0, L)] & 3) * D
            ncol = (nidx[pl.ds(row0, L)] & 3) * D
            accp = jnp.zeros((L,), jnp.float32)
            accn = jnp.zeros((L,), jnp.float32)
            for d in range(D):
                uv = plsc.load_gather(ubuf, [ridx, ucol + d])
                pv = plsc.load_gather(pbuf, [ridx, pcol + d])
                nv = plsc.load_gather(nbuf, [ridx, ncol + d])
                accp = accp + uv * pv
                accn = accn + uv * nv
            pout[pl.ds(row0, L)] = accp
            nout[pl.ds(row0, L)] = accn
        return carry

    lax.fori_loop(0, NCHUNK, chunk_body, 0)

    pltpu.sync_copy(pout, pout_hbm.at[pl.ds(base, BPW)])
    pltpu.sync_copy(nout, nout_hbm.at[pl.ds(base, BPW)])


def kernel(users, positive_items, negative_items, user_table, item_table):
    u = users.astype(jnp.int32).reshape(NW, BPW)
    p = positive_items.astype(jnp.int32).reshape(NW, BPW)
    n = negative_items.astype(jnp.int32).reshape(NW, BPW)
    ut = user_table.reshape(VP, 128)
    it = item_table.reshape(VP, 128)
    return _mf_kernel(u, p, n, ut, it)
